# Initial kernel scaffold; baseline (speedup 1.0000x reference)
import jax
import jax.numpy as jnp
from jax.experimental import pallas as pl


def kernel(X_vals, weight, log_alpha, bias, X_row, X_col):
    raise NotImplementedError("write your pallas kernel here")


# speedup vs baseline: 169.2712x; 169.2712x over previous
import functools

import jax
import jax.numpy as jnp
from jax import lax
from jax.experimental import pallas as pl
from jax.experimental.pallas import tpu as pltpu
from jax.experimental.pallas import tpu_sc as plsc

N_SAMPLES = 65536
N_FEATURES = 65536
BETA = 2.0 / 3.0
GAMMA = -0.1
ZETA = 1.1

NC = 2
NS = 16
NW = NC * NS

CHUNK = 2048
ROWS_PER_CHUNK = CHUNK // 128
FPW = N_FEATURES // NS


def _gate(log_alpha):
    s = 1.0 / (1.0 + jnp.exp(log_alpha * (-1.0 / BETA)))
    s_bar = s * (ZETA - GAMMA) + GAMMA
    return jnp.minimum(jnp.maximum(s_bar, 0.0), 1.0)


def _sc_partials(n_chunks):
    mesh = plsc.VectorSubcoreMesh(
        core_axis_name="c", subcore_axis_name="s", num_cores=NC, num_subcores=NS
    )

    @functools.partial(
        pl.kernel,
        out_type=jax.ShapeDtypeStruct((NC, N_SAMPLES), jnp.float32),
        mesh=mesh,
        compiler_params=pltpu.CompilerParams(needs_layout_passes=False),
        scratch_types=dict(
            gw_local=pltpu.VMEM((N_FEATURES,), jnp.float32),
            fa=pltpu.VMEM((FPW,), jnp.float32),
            fb=pltpu.VMEM((FPW,), jnp.float32),
            colsb=pltpu.VMEM((CHUNK,), jnp.int32),
            valsb=pltpu.VMEM((CHUNK,), jnp.float32),
            rowsb=pltpu.VMEM((ROWS_PER_CHUNK, 128), jnp.int32),
            contribb=pltpu.VMEM((ROWS_PER_CHUNK, 128), jnp.float32),
            gw_shared=pltpu.VMEM_SHARED((N_FEATURES,), jnp.float32),
            y_shared=pltpu.VMEM_SHARED((N_SAMPLES,), jnp.float32),
        ),
    )
    def body(vals_hbm, w_hbm, la_hbm, cols_hbm, rows2d_hbm, out_hbm,
             gw_local, fa, fb, colsb, valsb, rowsb, contribb,
             gw_shared, y_shared):
        cid = lax.axis_index("c")
        sid = lax.axis_index("s")
        fbase = sid * FPW

        pltpu.sync_copy(w_hbm.at[pl.ds(fbase, FPW)], fa)
        pltpu.sync_copy(la_hbm.at[pl.ds(fbase, FPW)], fb)

        def gate_body(g, _):
            off = g * 16
            w = fa[pl.ds(off, 16)]
            la = fb[pl.ds(off, 16)]
            fb[pl.ds(off, 16)] = w * _gate(la)
            fa[pl.ds(off, 16)] = jnp.zeros((16,), jnp.float32)
            return 0

        lax.fori_loop(0, FPW // 16, gate_body, 0)
        pltpu.sync_copy(fb, gw_shared.at[pl.ds(fbase, FPW)])
        pltpu.sync_copy(fa, y_shared.at[pl.ds(fbase, FPW)])
        plsc.subcore_barrier()
        pltpu.sync_copy(gw_shared, gw_local)

        wid = cid * NS + sid
        nbase = wid * n_chunks * CHUNK
        rbase0 = wid * n_chunks * ROWS_PER_CHUNK

        def chunk_body(t, _):
            base = nbase + t * CHUNK
            pltpu.sync_copy(cols_hbm.at[pl.ds(base, CHUNK)], colsb)
            pltpu.sync_copy(vals_hbm.at[pl.ds(base, CHUNK)], valsb)
            pltpu.sync_copy(
                rows2d_hbm.at[pl.ds(rbase0 + t * ROWS_PER_CHUNK, ROWS_PER_CHUNK)],
                rowsb,
            )

            def row_body(j, _):
                for k in range(8):
                    off = j * 128 + k * 16
                    cv = colsb[pl.ds(off, 16)]
                    vv = valsb[pl.ds(off, 16)]
                    gw = plsc.load_gather(gw_local, [cv])
                    contribb[j, pl.ds(k * 16, 16)] = vv * gw
                return 0

            lax.fori_loop(0, ROWS_PER_CHUNK, row_body, 0)
            for j in range(ROWS_PER_CHUNK):
                pltpu.sync_copy(
                    contribb.at[j], y_shared.at[rowsb.at[j]], add=True
                )
            return 0

        lax.fori_loop(0, n_chunks, chunk_body, 0)

        plsc.subcore_barrier()
        pltpu.sync_copy(y_shared.at[pl.ds(fbase, FPW)], fa)
        pltpu.sync_copy(fa, out_hbm.at[cid, pl.ds(fbase, FPW)])

    return body


def _combine_body(p_ref, b_ref, o_ref):
    o_ref[...] = p_ref[0] + p_ref[1] + b_ref[0, 0]


def kernel(X_vals, weight, log_alpha, bias, X_row, X_col):
    nnz = X_vals.shape[0]
    n_chunks = -(-nnz // (NW * CHUNK))
    padn = NW * n_chunks * CHUNK
    pad = padn - nnz

    vals = jnp.pad(X_vals, (0, pad))
    cols = jnp.pad(X_col.astype(jnp.int32), (0, pad))
    rows2d = jnp.pad(X_row.astype(jnp.int32), (0, pad)).reshape(padn // 128, 128)

    partials = _sc_partials(n_chunks)(vals, weight, log_alpha, cols, rows2d)

    y2d = pl.pallas_call(
        _combine_body,
        out_shape=jax.ShapeDtypeStruct((N_SAMPLES // 128, 128), jnp.float32),
        in_specs=[
            pl.BlockSpec(memory_space=pltpu.VMEM),
            pl.BlockSpec(memory_space=pltpu.SMEM),
        ],
        out_specs=pl.BlockSpec(memory_space=pltpu.VMEM),
    )(partials.reshape(NC, N_SAMPLES // 128, 128), bias.reshape(1, 1))
    return y2d.reshape(N_SAMPLES)


# speedup vs baseline: 187.4112x; 1.1072x over previous
import functools

import jax
import jax.numpy as jnp
from jax import lax
from jax.experimental import pallas as pl
from jax.experimental.pallas import tpu as pltpu
from jax.experimental.pallas import tpu_sc as plsc

N_SAMPLES = 65536
N_FEATURES = 65536
BETA = 2.0 / 3.0
GAMMA = -0.1
ZETA = 1.1

NC = 2
NS = 16
NW = NC * NS

NBUF = 4
CHUNK = 2048
RPC = CHUNK // 128
FPW = N_FEATURES // NS
FPG = N_FEATURES // NW

_MESH = plsc.VectorSubcoreMesh(
    core_axis_name="c", subcore_axis_name="s", num_cores=NC, num_subcores=NS
)
_PARAMS = pltpu.CompilerParams(needs_layout_passes=False)


def _gate(log_alpha):
    s = 1.0 / (1.0 + jnp.exp(log_alpha * (-1.0 / BETA)))
    s_bar = s * (ZETA - GAMMA) + GAMMA
    return jnp.minimum(jnp.maximum(s_bar, 0.0), 1.0)


@functools.partial(
    pl.kernel,
    out_type=jax.ShapeDtypeStruct((N_FEATURES,), jnp.float32),
    mesh=_MESH,
    compiler_params=_PARAMS,
    scratch_types=dict(
        wa=pltpu.VMEM((FPG,), jnp.float32),
        la=pltpu.VMEM((FPG,), jnp.float32),
    ),
)
def _gw_kernel(w_hbm, la_hbm, gw_hbm, wa, la):
    wid = lax.axis_index("c") * NS + lax.axis_index("s")
    fbase = wid * FPG
    pltpu.sync_copy(w_hbm.at[pl.ds(fbase, FPG)], wa)
    pltpu.sync_copy(la_hbm.at[pl.ds(fbase, FPG)], la)

    def body(g, _):
        off = g * 16
        wa[pl.ds(off, 16)] = wa[pl.ds(off, 16)] * _gate(la[pl.ds(off, 16)])
        return 0

    lax.fori_loop(0, FPG // 16, body, 0)
    pltpu.sync_copy(wa, gw_hbm.at[pl.ds(fbase, FPG)])


def _main_kernel(n_chunks):
    assert n_chunks % NBUF == 0

    @functools.partial(
        pl.kernel,
        out_type=jax.ShapeDtypeStruct((NC, N_SAMPLES), jnp.float32),
        mesh=_MESH,
        compiler_params=_PARAMS,
        scratch_types=dict(
            gw_local=pltpu.VMEM((N_FEATURES,), jnp.float32),
            colsb=pltpu.VMEM((NBUF, CHUNK), jnp.int32),
            valsb=pltpu.VMEM((NBUF, CHUNK), jnp.float32),
            rowsb=pltpu.VMEM((NBUF, RPC, 128), jnp.int32),
            contribb=pltpu.VMEM((NBUF, RPC, 128), jnp.float32),
            ybuf=pltpu.VMEM((FPW,), jnp.float32),
            y_shared=pltpu.VMEM_SHARED((N_SAMPLES,), jnp.float32),
            sem_gw=pltpu.SemaphoreType.DMA,
            sem_in0=pltpu.SemaphoreType.DMA,
            sem_in1=pltpu.SemaphoreType.DMA,
            sem_in2=pltpu.SemaphoreType.DMA,
            sem_in3=pltpu.SemaphoreType.DMA,
            sem_sc0=pltpu.SemaphoreType.DMA,
            sem_sc1=pltpu.SemaphoreType.DMA,
            sem_sc2=pltpu.SemaphoreType.DMA,
            sem_sc3=pltpu.SemaphoreType.DMA,
        ),
    )
    def body(vals_hbm, gw_hbm, cols_hbm, rows2d_hbm, out_hbm,
             gw_local, colsb, valsb, rowsb, contribb, ybuf, y_shared,
             sem_gw, sem_in0, sem_in1, sem_in2, sem_in3,
             sem_sc0, sem_sc1, sem_sc2, sem_sc3):
        cid = lax.axis_index("c")
        sid = lax.axis_index("s")
        fbase = sid * FPW
        sem_in = (sem_in0, sem_in1, sem_in2, sem_in3)
        sem_sc = (sem_sc0, sem_sc1, sem_sc2, sem_sc3)

        gw_dma = pltpu.make_async_copy(gw_hbm, gw_local, sem_gw)
        gw_dma.start()

        def zero_body(j, _):
            z = jnp.zeros((16,), jnp.float32)
            zi = jnp.zeros((16,), jnp.int32)
            for b in range(NBUF):
                for k in range(8):
                    contribb[b, j, pl.ds(k * 16, 16)] = z
                    rowsb[b, j, pl.ds(k * 16, 16)] = zi
            return 0

        lax.fori_loop(0, RPC, zero_body, 0)

        def zero_y(g, _):
            ybuf[pl.ds(g * 16, 16)] = jnp.zeros((16,), jnp.float32)
            return 0

        lax.fori_loop(0, FPW // 16, zero_y, 0)
        pltpu.sync_copy(ybuf, y_shared.at[pl.ds(fbase, FPW)])
        plsc.subcore_barrier()

        wid = cid * NS + sid
        nbase = wid * n_chunks * CHUNK
        rbase0 = wid * n_chunks * RPC

        def start_inputs(t, b):
            base = nbase + t * CHUNK
            pltpu.make_async_copy(
                cols_hbm.at[pl.ds(base, CHUNK)], colsb.at[b], sem_in[b]
            ).start()
            pltpu.make_async_copy(
                vals_hbm.at[pl.ds(base, CHUNK)], valsb.at[b], sem_in[b]
            ).start()
            pltpu.make_async_copy(
                rows2d_hbm.at[pl.ds(rbase0 + t * RPC, RPC)], rowsb.at[b],
                sem_in[b],
            ).start()

        def wait_inputs(t, b):
            base = nbase + t * CHUNK
            pltpu.make_async_copy(
                cols_hbm.at[pl.ds(base, CHUNK)], colsb.at[b], sem_in[b]
            ).wait()
            pltpu.make_async_copy(
                vals_hbm.at[pl.ds(base, CHUNK)], valsb.at[b], sem_in[b]
            ).wait()
            pltpu.make_async_copy(
                rows2d_hbm.at[pl.ds(rbase0 + t * RPC, RPC)], rowsb.at[b],
                sem_in[b],
            ).wait()

        def fire_scatters(b):
            for j in range(RPC):
                pltpu.make_async_copy(
                    contribb.at[b, j], y_shared.at[rowsb.at[b, j]], sem_sc[b]
                ).start(add=True)

        def drain_scatters(b):
            for j in range(RPC):
                pltpu.make_async_copy(
                    contribb.at[b, j], y_shared.at[rowsb.at[b, j]], sem_sc[b]
                ).wait()

        start_inputs(0, 0)
        start_inputs(1, 1)
        fire_scatters(2)
        fire_scatters(3)
        gw_dma.wait()

        def chunk_quad(i, _):
            for b in range(NBUF):
                t = i * NBUF + b
                wait_inputs(t, b)

                def row_body(j, _):
                    for k in range(8):
                        off = j * 128 + k * 16
                        cv = colsb[b, pl.ds(off, 16)]
                        vv = valsb[b, pl.ds(off, 16)]
                        gw = plsc.load_gather(gw_local, [cv])
                        contribb[b, j, pl.ds(k * 16, 16)] = vv * gw
                    return 0

                lax.fori_loop(0, RPC, row_body, 0)
                fire_scatters(b)

                bd = (b + 2) % NBUF
                drain_scatters(bd)

                @pl.when(t + 2 < n_chunks)
                def _():
                    start_inputs(t + 2, bd)
            return 0

        lax.fori_loop(0, n_chunks // NBUF, chunk_quad, 0)
        drain_scatters((n_chunks - 2) % NBUF)
        drain_scatters((n_chunks - 1) % NBUF)

        plsc.subcore_barrier()
        pltpu.sync_copy(y_shared.at[pl.ds(fbase, FPW)], ybuf)
        pltpu.sync_copy(ybuf, out_hbm.at[cid, pl.ds(fbase, FPW)])

    return body


def _combine_body(p_ref, b_ref, o_ref):
    o_ref[...] = p_ref[0] + p_ref[1] + b_ref[0, 0]


def kernel(X_vals, weight, log_alpha, bias, X_row, X_col):
    nnz = X_vals.shape[0]
    n_chunks = -(-nnz // (NW * CHUNK))
    n_chunks += (-n_chunks) % NBUF
    padn = NW * n_chunks * CHUNK
    pad = padn - nnz

    vals = jnp.pad(X_vals, (0, pad))
    cols = jnp.pad(X_col.astype(jnp.int32), (0, pad))
    rows2d = jnp.pad(X_row.astype(jnp.int32), (0, pad)).reshape(padn // 128, 128)

    gw = _gw_kernel(weight, log_alpha)
    partials = _main_kernel(n_chunks)(vals, gw, cols, rows2d)

    y2d = pl.pallas_call(
        _combine_body,
        out_shape=jax.ShapeDtypeStruct((N_SAMPLES // 128, 128), jnp.float32),
        in_specs=[
            pl.BlockSpec(memory_space=pltpu.VMEM),
            pl.BlockSpec(memory_space=pltpu.SMEM),
        ],
        out_specs=pl.BlockSpec(memory_space=pltpu.VMEM),
    )(partials.reshape(NC, N_SAMPLES // 128, 128), bias.reshape(1, 1))
    return y2d.reshape(N_SAMPLES)
